# Initial kernel scaffold; baseline (speedup 1.0000x reference)
import jax
import jax.numpy as jnp
from jax.experimental import pallas as pl


def kernel(hidden_states, router_weight, gate_w, up_w, down_w, shared_gate_w, shared_up_w, shared_down_w):
    raise NotImplementedError("write your pallas kernel here")


# speedup vs baseline: 4.6003x; 4.6003x over previous
import functools

import jax
import jax.numpy as jnp
from jax import lax
from jax.experimental import pallas as pl
from jax.experimental.pallas import tpu as pltpu
from jax.experimental.pallas import tpu_sc as plsc

T = 2048
D = 1024
E = 64
TOPK = 2
DFF = 512
SDFF = 1024
SCALE = 1.0

EE = E + 2
N_R = T * TOPK
N_ALL = N_R + 2 * T

BT = 256
NB = N_ALL // BT
NBR = N_R // BT
SHBLK = T // BT
G = NB + EE - 1

SC_CORES = 2
SC_SUBCORES = 16
NW = SC_CORES * SC_SUBCORES
GCH = 64
TT = 16


def _gating_body(x_ref, rw_ref, vals_ref, idx_ref):
    x = x_ref[...]
    rw = rw_ref[...]
    logits = lax.dot_general(x, rw, (((1,), (1,)), ((), ())),
                             preferred_element_type=jnp.float32)
    m = jnp.max(logits, axis=1, keepdims=True)
    ex = jnp.exp(logits - m)
    scores = ex / jnp.sum(ex, axis=1, keepdims=True)
    col = lax.broadcasted_iota(jnp.int32, scores.shape, 1)
    v1 = jnp.max(scores, axis=1)
    i1 = jnp.min(jnp.where(scores == v1[:, None], col, E), axis=1)
    s2 = jnp.where(col == i1[:, None], -jnp.inf, scores)
    v2 = jnp.max(s2, axis=1)
    i2 = jnp.min(jnp.where(s2 == v2[:, None], col, E), axis=1)
    vals_ref[...] = jnp.stack([v1, v2], axis=1)
    idx_ref[...] = jnp.stack([i1, i2], axis=1)


def _gating(hidden, router_weight):
    bt = 256
    return pl.pallas_call(
        _gating_body,
        grid=(T // bt,),
        in_specs=[
            pl.BlockSpec((bt, D), lambda i: (i, 0)),
            pl.BlockSpec((E, D), lambda i: (0, 0)),
        ],
        out_specs=[
            pl.BlockSpec((bt, TOPK), lambda i: (i, 0)),
            pl.BlockSpec((bt, TOPK), lambda i: (i, 0)),
        ],
        out_shape=[
            jax.ShapeDtypeStruct((T, TOPK), jnp.float32),
            jax.ShapeDtypeStruct((T, TOPK), jnp.int32),
        ],
    )(hidden, router_weight)


@functools.lru_cache(maxsize=None)
def _make_sc_gather():
    mesh = plsc.VectorSubcoreMesh(core_axis_name="c", subcore_axis_name="s")

    @functools.partial(
        pl.kernel,
        mesh=mesh,
        out_type=jax.ShapeDtypeStruct((N_R, D), jnp.float32),
        scratch_types=[
            pltpu.VMEM((GCH,), jnp.int32),
            pltpu.VMEM((GCH, D), jnp.float32),
            pltpu.SemaphoreType.DMA,
        ],
    )
    def _sc_gather_kernel(hid_hbm, idx_hbm, out_hbm, idx_v, rows_v, sem):
        wid = lax.axis_index("s") * SC_CORES + lax.axis_index("c")
        n_chunks = N_R // GCH // NW
        for j in range(n_chunks):
            base = (wid * n_chunks + j) * GCH
            pltpu.sync_copy(idx_hbm.at[pl.ds(base, GCH)], idx_v)
            pltpu.async_copy(hid_hbm.at[idx_v], rows_v, sem).wait()
            pltpu.sync_copy(rows_v, out_hbm.at[pl.ds(base, GCH)])

    return _sc_gather_kernel


def _dispatch_gather(hidden, sorted_tok):
    return _make_sc_gather()(hidden, sorted_tok)


def _expert_body(ve, vm, vf, vv,
                 xs_ref, hid_ref, gr_ref, ur_ref, dr_ref,
                 gs_ref, us_ref, ds_ref, se_ref, sw_ref, y_ref):
    g = pl.program_id(0)
    e = ve[g]
    valid = vv[g] == 1
    first = vf[g] == 1
    is_routed = e < E

    x = jnp.where(is_routed, xs_ref[...], hid_ref[...])
    gw = jnp.where(is_routed, gr_ref[0], gs_ref[0])
    uw = jnp.where(is_routed, ur_ref[0], us_ref[0])
    dw = jnp.where(is_routed, dr_ref[0], ds_ref[0])

    h1 = lax.dot_general(x, gw, (((1,), (1,)), ((), ())),
                         preferred_element_type=jnp.float32)
    h2 = lax.dot_general(x, uw, (((1,), (1,)), ((), ())),
                         preferred_element_type=jnp.float32)
    h = (h1 * jax.nn.sigmoid(h1)) * h2
    y = lax.dot_general(h, dw, (((1,), (1,)), ((), ())),
                        preferred_element_type=jnp.float32)

    w = jnp.where(se_ref[0, 0, :] == e, sw_ref[0, 0, :], 0.0)
    contrib = y * w[:, None]

    @pl.when(valid & first)
    def _():
        y_ref[...] = contrib

    @pl.when(valid & jnp.logical_not(first))
    def _():
        y_ref[...] += contrib


def _grouped_mlp(xs, hidden, gate_w, up_w, down_w, sgate2, sup2, sdown2,
                 se3, sw3, visit_e, visit_m, visit_f, visit_v):
    def _xs_idx(g, ve, vm, vf, vv):
        return (jnp.minimum(vm[g], NBR - 1), 0)

    def _hid_idx(g, ve, vm, vf, vv):
        b = vm[g]
        h = jnp.where(b >= NBR + SHBLK, b - NBR - SHBLK, b - NBR)
        return (jnp.clip(h, 0, SHBLK - 1), 0)

    def _wr_idx(g, ve, vm, vf, vv):
        return (jnp.minimum(ve[g], E - 1), 0, 0)

    def _ws_idx(g, ve, vm, vf, vv):
        return (jnp.clip(ve[g] - E, 0, 1), 0, 0)

    def _row_idx(g, ve, vm, vf, vv):
        return (vm[g], 0, 0)

    def _out_idx(g, ve, vm, vf, vv):
        return (vm[g], 0)

    grid_spec = pltpu.PrefetchScalarGridSpec(
        num_scalar_prefetch=4,
        grid=(G,),
        in_specs=[
            pl.BlockSpec((BT, D), _xs_idx),
            pl.BlockSpec((BT, D), _hid_idx),
            pl.BlockSpec((1, DFF, D), _wr_idx),
            pl.BlockSpec((1, DFF, D), _wr_idx),
            pl.BlockSpec((1, D, DFF), _wr_idx),
            pl.BlockSpec((1, DFF, D), _ws_idx),
            pl.BlockSpec((1, DFF, D), _ws_idx),
            pl.BlockSpec((1, D, DFF), _ws_idx),
            pl.BlockSpec((1, 1, BT), _row_idx),
            pl.BlockSpec((1, 1, BT), _row_idx),
        ],
        out_specs=pl.BlockSpec((BT, D), _out_idx),
    )
    return pl.pallas_call(
        _expert_body,
        grid_spec=grid_spec,
        out_shape=jax.ShapeDtypeStruct((N_ALL, D), jnp.float32),
    )(visit_e, visit_m, visit_f, visit_v,
      xs, hidden, gate_w, up_w, down_w, sgate2, sup2, sdown2, se3, sw3)


@functools.lru_cache(maxsize=None)
def _make_sc_combine():
    mesh = plsc.VectorSubcoreMesh(core_axis_name="c", subcore_axis_name="s")

    @functools.partial(
        pl.kernel,
        mesh=mesh,
        out_type=jax.ShapeDtypeStruct((T, D), jnp.float32),
        scratch_types=[
            pltpu.VMEM((4 * TT,), jnp.int32),
            pltpu.VMEM((4 * TT, D), jnp.float32),
            pltpu.VMEM((TT, D), jnp.float32),
            pltpu.SemaphoreType.DMA,
        ],
    )
    def _sc_combine_kernel(y_hbm, pos_hbm, out_hbm, idx_v, rows_v, acc_v, sem):
        wid = lax.axis_index("s") * SC_CORES + lax.axis_index("c")
        tok_per_w = T // NW
        n_chunks = tok_per_w // TT
        for j in range(n_chunks):
            tok0 = wid * tok_per_w + j * TT
            pltpu.sync_copy(pos_hbm.at[pl.ds(tok0 * 4, 4 * TT)], idx_v)
            pltpu.async_copy(y_hbm.at[idx_v], rows_v, sem).wait()

            def col_body(ci, _):
                off = ci * 16
                for tt in range(TT):
                    s = (rows_v[4 * tt + 0, pl.ds(off, 16)]
                         + rows_v[4 * tt + 1, pl.ds(off, 16)]
                         + rows_v[4 * tt + 2, pl.ds(off, 16)]
                         + rows_v[4 * tt + 3, pl.ds(off, 16)])
                    acc_v[tt, pl.ds(off, 16)] = s
                return 0

            lax.fori_loop(0, D // 16, col_body, 0)
            pltpu.sync_copy(acc_v, out_hbm.at[pl.ds(tok0, TT)])

    return _sc_combine_kernel


def _combine(y, pos):
    return _make_sc_combine()(y, pos)


def kernel(hidden_states, router_weight, gate_w, up_w, down_w,
           shared_gate_w, shared_up_w, shared_down_w):
    vals, idx = _gating(hidden_states, router_weight)

    e_flat = idx.reshape(-1)
    sidx = jnp.argsort(e_flat, stable=True).astype(jnp.int32)
    sorted_e_r = jnp.take(e_flat, sidx)
    sorted_tok = (sidx // TOPK).astype(jnp.int32)
    sorted_w_r = jnp.take(vals.reshape(-1), sidx) * SCALE
    inv = jnp.argsort(sidx).astype(jnp.int32)

    sorted_e = jnp.concatenate([
        sorted_e_r,
        jnp.full((T,), E, dtype=jnp.int32),
        jnp.full((T,), E + 1, dtype=jnp.int32),
    ])
    sorted_w = jnp.concatenate([
        sorted_w_r, jnp.ones((2 * T,), dtype=jnp.float32)])

    offs_r = jnp.searchsorted(sorted_e_r, jnp.arange(E + 1), side="left")
    off = jnp.concatenate(
        [offs_r, jnp.array([N_R + T, N_ALL])]).astype(jnp.int32)
    cnt = off[1:] - off[:-1]
    first_blk = off[:-1] // BT
    last_blk = (off[1:] - 1) // BT
    nb_e = jnp.where(cnt > 0, last_blk - first_blk + 1, 0)
    vcum = jnp.cumsum(nb_e)
    total = vcum[EE - 1]
    garr = jnp.arange(G)
    eg = jnp.minimum(jnp.searchsorted(vcum, garr, side="right"), EE - 1)
    prev = vcum[eg] - nb_e[eg]
    m = first_blk[eg] + (garr - prev)
    validv = garr < total
    visit_e = jnp.where(validv, eg, EE - 1).astype(jnp.int32)
    visit_m = jnp.where(validv, m, NB - 1).astype(jnp.int32)
    visit_f = (jnp.concatenate([
        jnp.array([1], dtype=jnp.int32),
        (visit_m[1:] != visit_m[:-1]).astype(jnp.int32)])
        * validv.astype(jnp.int32))
    visit_v = validv.astype(jnp.int32)

    xs = _dispatch_gather(hidden_states, sorted_tok)

    sgate2 = shared_gate_w.reshape(2, DFF, D)
    sup2 = shared_up_w.reshape(2, DFF, D)
    sdown2 = shared_down_w.reshape(D, 2, DFF).transpose(1, 0, 2)
    se3 = sorted_e.reshape(NB, 1, BT)
    sw3 = sorted_w.reshape(NB, 1, BT)

    y = _grouped_mlp(xs, hidden_states, gate_w, up_w, down_w,
                     sgate2, sup2, sdown2, se3, sw3,
                     visit_e, visit_m, visit_f, visit_v)

    tarr = jnp.arange(T, dtype=jnp.int32)
    pos = jnp.stack([
        inv.reshape(T, TOPK)[:, 0],
        inv.reshape(T, TOPK)[:, 1],
        N_R + tarr,
        N_R + T + tarr,
    ], axis=1).reshape(-1)

    return _combine(y, pos)
